# baseline (speedup 1.0000x reference)
import functools
import math

import jax
import jax.numpy as jnp
from jax import lax
from jax.experimental import pallas as pl
from jax.experimental.pallas import tpu as pltpu
from jax.experimental.pallas import tpu_sc as plsc

D_MODEL = 64
SCALE = math.sqrt(D_MODEL)
NUM_CORES = 2
NUM_SUBCORES = 16
NW = NUM_CORES * NUM_SUBCORES
CHUNK = 128
NBUF = 4
L = 16


@functools.lru_cache(maxsize=None)
def _make_embed(nbatch: int, seq: int, vocab: int):
    assert nbatch == NW * CHUNK and seq % NBUF == 0
    n_groups = seq // NBUF
    mesh = plsc.VectorSubcoreMesh(core_axis_name="c", subcore_axis_name="s")

    @functools.partial(
        pl.kernel,
        mesh=mesh,
        compiler_params=pltpu.CompilerParams(
            use_tc_tiling_on_sc=False, needs_layout_passes=False
        ),
        out_type=jax.ShapeDtypeStruct((seq, D_MODEL, nbatch), jnp.float32),
        scratch_types=(
            [pltpu.VMEM((seq, CHUNK), jnp.int32)]
            + [pltpu.VMEM((CHUNK, D_MODEL), jnp.float32) for _ in range(NBUF)]
            + [pltpu.VMEM((D_MODEL, CHUNK), jnp.float32) for _ in range(NBUF)]
            + [pltpu.SemaphoreType.DMA((NBUF,)), pltpu.SemaphoreType.DMA((NBUF,))]
        ),
    )
    def embed(xt_hbm, lut_hbm, out_hbm, idx_v, *rest):
        gbuf = rest[:NBUF]
        wbuf = rest[NBUF : 2 * NBUF]
        gsem, wsem = rest[2 * NBUF], rest[2 * NBUF + 1]
        wid = lax.axis_index("s") * NUM_CORES + lax.axis_index("c")
        b0 = wid * CHUNK
        pltpu.sync_copy(xt_hbm.at[:, pl.ds(b0, CHUNK)], idx_v)

        for b in range(NBUF):
            pltpu.async_copy(lut_hbm.at[idx_v.at[b]], gbuf[b], gsem.at[b])

        def group_body(cc, carry):
            for b in range(NBUF):
                s = cc * NBUF + b
                pltpu.make_async_copy(
                    lut_hbm.at[idx_v.at[0]], gbuf[b], gsem.at[b]
                ).wait()

                @pl.when(cc > 0)
                def _wait_wb():
                    pltpu.make_async_copy(
                        wbuf[b], out_hbm.at[0, :, pl.ds(0, CHUNK)], wsem.at[b]
                    ).wait()

                def d_body(d, carry2):
                    col = jnp.full((L,), 0, jnp.int32) + d
                    for k in range(CHUNK // L):
                        rows = lax.iota(jnp.int32, L) + (k * L)
                        v = plsc.load_gather(gbuf[b], [rows, col])
                        wbuf[b][d, pl.ds(k * L, L)] = v * SCALE
                    return carry2

                lax.fori_loop(0, D_MODEL, d_body, 0)

                pltpu.async_copy(
                    wbuf[b],
                    out_hbm.at[s, :, pl.ds(b0, CHUNK)],
                    wsem.at[b],
                )

                @pl.when(cc < n_groups - 1)
                def _next_gather():
                    pltpu.async_copy(
                        lut_hbm.at[idx_v.at[s + NBUF]], gbuf[b], gsem.at[b]
                    )

            return carry

        lax.fori_loop(0, n_groups, group_body, 0)

        for b in range(NBUF):
            pltpu.make_async_copy(
                wbuf[b], out_hbm.at[0, :, pl.ds(0, CHUNK)], wsem.at[b]
            ).wait()

    return embed


def kernel(x, lut):
    nb, seq = x.shape
    xt = jnp.transpose(x).astype(jnp.int32)
    out_t = _make_embed(nb, seq, lut.shape[0])(xt, lut)
    return jnp.transpose(out_t, (2, 0, 1))


# speedup vs baseline: 1.6216x; 1.6216x over previous
import functools
import math

import jax
import jax.numpy as jnp
from jax import lax
from jax.experimental import pallas as pl
from jax.experimental.pallas import tpu as pltpu
from jax.experimental.pallas import tpu_sc as plsc

D_MODEL = 64
SCALE = math.sqrt(D_MODEL)
NUM_CORES = 2
NUM_SUBCORES = 16
NW = NUM_CORES * NUM_SUBCORES
CHUNK = 128
NBUF = 4


@functools.lru_cache(maxsize=None)
def _make_embed(nbatch: int, seq: int, vocab: int):
    assert nbatch == NW * CHUNK and seq % NBUF == 0
    n_groups = seq // NBUF
    mesh = plsc.VectorSubcoreMesh(core_axis_name="c", subcore_axis_name="s")

    @functools.partial(
        pl.kernel,
        mesh=mesh,
        compiler_params=pltpu.CompilerParams(use_tc_tiling_on_sc=False),
        out_type=jax.ShapeDtypeStruct((seq, nbatch, D_MODEL), jnp.float32),
        scratch_types=[
            pltpu.VMEM((seq, CHUNK), jnp.int32),
            pltpu.VMEM((NBUF, CHUNK, D_MODEL), jnp.float32),
            pltpu.VMEM((NBUF, CHUNK, D_MODEL), jnp.float32),
            pltpu.SemaphoreType.DMA((NBUF,)),
            pltpu.SemaphoreType.DMA((NBUF,)),
        ],
    )
    def embed(xt_hbm, lut_hbm, out_hbm, idx_v, gbuf, wbuf, gsem, wsem):
        wid = lax.axis_index("s") * NUM_CORES + lax.axis_index("c")
        b0 = wid * CHUNK
        pltpu.sync_copy(xt_hbm.at[:, pl.ds(b0, CHUNK)], idx_v)

        for b in range(NBUF):
            pltpu.async_copy(lut_hbm.at[idx_v.at[b]], gbuf.at[b], gsem.at[b])

        def group_body(cc, carry):
            for b in range(NBUF):
                s = cc * NBUF + b
                pltpu.make_async_copy(
                    lut_hbm.at[idx_v.at[0]], gbuf.at[b], gsem.at[b]
                ).wait()

                @pl.when(cc > 0)
                def _wait_wb():
                    pltpu.make_async_copy(
                        wbuf.at[b], out_hbm.at[0, pl.ds(0, CHUNK)], wsem.at[b]
                    ).wait()

                def row_body(r, carry2):
                    for j in range(D_MODEL // 16):
                        sl = pl.ds(j * 16, 16)
                        wbuf[b, r, sl] = gbuf[b, r, sl] * SCALE
                    return carry2

                lax.fori_loop(0, CHUNK, row_body, 0, unroll=4)

                pltpu.async_copy(
                    wbuf.at[b],
                    out_hbm.at[s, pl.ds(b0, CHUNK)],
                    wsem.at[b],
                )

                @pl.when(cc < n_groups - 1)
                def _next_gather():
                    pltpu.async_copy(
                        lut_hbm.at[idx_v.at[s + NBUF]], gbuf.at[b], gsem.at[b]
                    )

            return carry

        lax.fori_loop(0, n_groups, group_body, 0)

        for b in range(NBUF):
            pltpu.make_async_copy(
                wbuf.at[b], out_hbm.at[0, pl.ds(0, CHUNK)], wsem.at[b]
            ).wait()

    return embed


def kernel(x, lut):
    nb, seq = x.shape
    xt = jnp.transpose(x).astype(jnp.int32)
    out_t = _make_embed(nb, seq, lut.shape[0])(xt, lut)
    return jnp.transpose(out_t, (1, 0, 2))


# speedup vs baseline: 1.7270x; 1.0650x over previous
import functools
import math

import jax
import jax.numpy as jnp
from jax import lax
from jax.experimental import pallas as pl
from jax.experimental.pallas import tpu as pltpu
from jax.experimental.pallas import tpu_sc as plsc

D_MODEL = 64
SCALE = math.sqrt(D_MODEL)
NUM_CORES = 2
NUM_SUBCORES = 16
NW = NUM_CORES * NUM_SUBCORES
CHUNK = 128
NBUF = 2
L = 16


@functools.lru_cache(maxsize=None)
def _make_embed(nbatch: int, seq: int, vocab2: int):
    assert nbatch == NW * CHUNK and seq % NBUF == 0
    n_groups = seq // NBUF
    mesh = plsc.VectorSubcoreMesh(core_axis_name="c", subcore_axis_name="s")

    @functools.partial(
        pl.kernel,
        mesh=mesh,
        compiler_params=pltpu.CompilerParams(use_tc_tiling_on_sc=True),
        out_type=jax.ShapeDtypeStruct((seq, nbatch, D_MODEL), jnp.float32),
        scratch_types=[
            pltpu.VMEM((seq, CHUNK), jnp.int32),
            pltpu.VMEM((NBUF, CHUNK), jnp.int32),
            pltpu.VMEM((NBUF, CHUNK, 2 * D_MODEL), jnp.float32),
            pltpu.VMEM((NBUF, CHUNK, D_MODEL), jnp.float32),
            pltpu.SemaphoreType.DMA((NBUF,)),
            pltpu.SemaphoreType.DMA((NBUF,)),
        ],
    )
    def embed(xt_hbm, lut_hbm, out_hbm, idx_v, half_v, gbuf, wbuf, gsem, wsem):
        wid = lax.axis_index("s") * NUM_CORES + lax.axis_index("c")
        b0 = wid * CHUNK
        pltpu.sync_copy(xt_hbm.at[:, pl.ds(b0, CHUNK)], idx_v)

        def start_gather(s, b):
            for m in range(CHUNK // L):
                sl = pl.ds(m * L, L)
                half_v[b, sl] = lax.shift_right_logical(idx_v[s, sl], 1)
            pltpu.async_copy(lut_hbm.at[half_v.at[b]], gbuf.at[b], gsem.at[b])

        for b in range(NBUF):
            start_gather(b, b)

        def group_body(cc, carry):
            for b in range(NBUF):
                s = cc * NBUF + b
                pltpu.make_async_copy(
                    lut_hbm.at[half_v.at[0]], gbuf.at[b], gsem.at[b]
                ).wait()

                @pl.when(cc > 0)
                def _wait_wb():
                    pltpu.make_async_copy(
                        wbuf.at[b], out_hbm.at[0, pl.ds(0, CHUNK)], wsem.at[b]
                    ).wait()

                def grp_body(m, carry2):
                    pv = idx_v[s, pl.ds(m * L, L)]
                    for ri in range(L):
                        r = m * L + ri
                        off = (pv[ri] & 1) * D_MODEL
                        for j in range(D_MODEL // L):
                            v = gbuf[b, r, pl.ds(off + j * L, L)]
                            wbuf[b, r, pl.ds(j * L, L)] = v * SCALE
                    return carry2

                lax.fori_loop(0, CHUNK // L, grp_body, 0)

                pltpu.async_copy(
                    wbuf.at[b],
                    out_hbm.at[s, pl.ds(b0, CHUNK)],
                    wsem.at[b],
                )

                @pl.when(cc < n_groups - 1)
                def _next_gather():
                    start_gather(s + NBUF, b)

            return carry

        lax.fori_loop(0, n_groups, group_body, 0)

        for b in range(NBUF):
            pltpu.make_async_copy(
                wbuf.at[b], out_hbm.at[0, pl.ds(0, CHUNK)], wsem.at[b]
            ).wait()

    return embed


def kernel(x, lut):
    nb, seq = x.shape
    xt = jnp.transpose(x).astype(jnp.int32)
    lut2 = lut.reshape(lut.shape[0] // 2, 2 * D_MODEL)
    out_t = _make_embed(nb, seq, lut2.shape[0])(xt, lut2)
    return jnp.transpose(out_t, (1, 0, 2))
